# Initial kernel scaffold; baseline (speedup 1.0000x reference)
import jax
import jax.numpy as jnp
from jax.experimental import pallas as pl


def kernel(gender, insurance, marital_status, race, language, scalar_inputs, emb_gender, emb_insurance, emb_marital_status, emb_race, emb_language, W1, b1, ln_g, ln_b, W2, b2):
    raise NotImplementedError("write your pallas kernel here")


# speedup vs baseline: 3.9364x; 3.9364x over previous
import functools

import jax
import jax.numpy as jnp
from jax import lax
from jax.experimental import pallas as pl
from jax.experimental.pallas import tpu as pltpu
from jax.experimental.pallas import tpu_sc as plsc

B = 16384
H = 128
O = 64
T1_ROWS = 3 * 6 * 6
T2_ROWS = 31 * 26

NC, NS = 2, 16
NW = NC * NS
BPW = B // NW
CHUNK = 128
NCH = BPW // CHUNK


def _tables_body(eg, ei, em, er, el, w1, b1, t1, t2):
    f32 = jnp.float32
    mg = jnp.dot(eg[...], w1[0:8, :], preferred_element_type=f32)
    mi = jnp.dot(ei[...], w1[8:16, :], preferred_element_type=f32)
    mm = jnp.dot(em[...], w1[16:24, :], preferred_element_type=f32)
    mr = jnp.dot(er[...], w1[24:32, :], preferred_element_type=f32)
    ml = jnp.dot(el[...], w1[32:40, :], preferred_element_type=f32)

    r1 = lax.broadcasted_iota(jnp.int32, (T1_ROWS, 1), 0)
    og = (r1 // 36 == lax.broadcasted_iota(jnp.int32, (T1_ROWS, 3), 1)).astype(f32)
    oi = ((r1 // 6) % 6 == lax.broadcasted_iota(jnp.int32, (T1_ROWS, 6), 1)).astype(f32)
    om = (r1 % 6 == lax.broadcasted_iota(jnp.int32, (T1_ROWS, 6), 1)).astype(f32)
    t1[...] = (jnp.dot(og, mg, preferred_element_type=f32)
               + jnp.dot(oi, mi, preferred_element_type=f32)
               + jnp.dot(om, mm, preferred_element_type=f32)
               + b1[...])

    r2 = lax.broadcasted_iota(jnp.int32, (T2_ROWS, 1), 0)
    orr = (r2 // 26 == lax.broadcasted_iota(jnp.int32, (T2_ROWS, 31), 1)).astype(f32)
    ol = (r2 % 26 == lax.broadcasted_iota(jnp.int32, (T2_ROWS, 26), 1)).astype(f32)
    t2[...] = (jnp.dot(orr, mr, preferred_element_type=f32)
               + jnp.dot(ol, ml, preferred_element_type=f32))


_build_tables = pl.pallas_call(
    _tables_body,
    out_shape=(jax.ShapeDtypeStruct((T1_ROWS, H), jnp.float32),
               jax.ShapeDtypeStruct((T2_ROWS, H), jnp.float32)),
)


def _gather_body(g, i, m, r, l, t1, t2, out,
                 gv, iv, mv, rv, lv, idx1, idx2, buf1, buf2, sem1, sem2):
    wid = lax.axis_index("s") * NC + lax.axis_index("c")
    base = wid * BPW

    pltpu.sync_copy(g.at[pl.ds(base, BPW)], gv)
    pltpu.sync_copy(i.at[pl.ds(base, BPW)], iv)
    pltpu.sync_copy(m.at[pl.ds(base, BPW)], mv)
    pltpu.sync_copy(r.at[pl.ds(base, BPW)], rv)
    pltpu.sync_copy(l.at[pl.ds(base, BPW)], lv)

    for k in range(BPW // 16):
        s = pl.ds(k * 16, 16)
        c = k // (CHUNK // 16)
        o = (k % (CHUNK // 16)) * 16
        idx1[c, pl.ds(o, 16)] = gv[s] * 36 + iv[s] * 6 + mv[s]
        idx2[c, pl.ds(o, 16)] = rv[s] * 26 + lv[s]

    for c in range(NCH):
        cp1 = pltpu.async_copy(t1.at[idx1.at[c]], buf1, sem1)
        cp2 = pltpu.async_copy(t2.at[idx2.at[c]], buf2, sem2)
        cp1.wait()
        cp2.wait()

        def row(rr, carry):
            for j in range(H // 16):
                s = pl.ds(j * 16, 16)
                plsc.addupdate(buf1.at[rr, s], buf2[rr, s])
            return carry
        lax.fori_loop(0, CHUNK, row, 0)

        pltpu.sync_copy(buf1, out.at[pl.ds(base + c * CHUNK, CHUNK)])


@functools.cache
def _make_gather_sum():
  return functools.partial(
    pl.kernel,
    out_type=jax.ShapeDtypeStruct((B, H), jnp.float32),
    mesh=plsc.VectorSubcoreMesh(core_axis_name="c", subcore_axis_name="s",
                                num_cores=NC, num_subcores=NS),
    scratch_types=[
        pltpu.VMEM((BPW,), jnp.int32),
        pltpu.VMEM((BPW,), jnp.int32),
        pltpu.VMEM((BPW,), jnp.int32),
        pltpu.VMEM((BPW,), jnp.int32),
        pltpu.VMEM((BPW,), jnp.int32),
        pltpu.VMEM((NCH, CHUNK), jnp.int32),
        pltpu.VMEM((NCH, CHUNK), jnp.int32),
        pltpu.VMEM((CHUNK, H), jnp.float32),
        pltpu.VMEM((CHUNK, H), jnp.float32),
        pltpu.SemaphoreType.DMA,
        pltpu.SemaphoreType.DMA,
    ],
  )(_gather_body)


def _mlp_body(hp, sc, w1r, lg, lb, w2, b2, out):
    x = hp[...] + sc[...] * w1r[...]
    mu = jnp.mean(x, axis=-1, keepdims=True)
    d = x - mu
    var = jnp.mean(d * d, axis=-1, keepdims=True)
    y = d * lax.rsqrt(var + 1e-5) * lg[...] + lb[...]
    y = jnp.maximum(y, 0.0)
    out[...] = jnp.dot(y, w2[...], preferred_element_type=jnp.float32) + b2[...]


_MLP_BS = 2048


def _mlp(hp, scal, w1row, ln_g, ln_b, W2, b2):
    grid = (B // _MLP_BS,)
    return pl.pallas_call(
        _mlp_body,
        grid=grid,
        in_specs=[
            pl.BlockSpec((_MLP_BS, H), lambda n: (n, 0)),
            pl.BlockSpec((_MLP_BS, 1), lambda n: (n, 0)),
            pl.BlockSpec((1, H), lambda n: (0, 0)),
            pl.BlockSpec((1, H), lambda n: (0, 0)),
            pl.BlockSpec((1, H), lambda n: (0, 0)),
            pl.BlockSpec((H, O), lambda n: (0, 0)),
            pl.BlockSpec((1, O), lambda n: (0, 0)),
        ],
        out_specs=pl.BlockSpec((_MLP_BS, O), lambda n: (n, 0)),
        out_shape=jax.ShapeDtypeStruct((B, O), jnp.float32),
    )(hp, scal, w1row, ln_g, ln_b, W2, b2)


def kernel(gender, insurance, marital_status, race, language, scalar_inputs,
           emb_gender, emb_insurance, emb_marital_status, emb_race, emb_language,
           W1, b1, ln_g, ln_b, W2, b2):
    g = gender.astype(jnp.int32)
    i = insurance.astype(jnp.int32)
    m = marital_status.astype(jnp.int32)
    r = race.astype(jnp.int32)
    l = language.astype(jnp.int32)

    t1, t2 = _build_tables(emb_gender, emb_insurance, emb_marital_status,
                           emb_race, emb_language, W1, b1.reshape(1, H))
    hp = _make_gather_sum()(g, i, m, r, l, t1, t2)
    return _mlp(hp, scalar_inputs, W1[40].reshape(1, H),
                ln_g.reshape(1, H), ln_b.reshape(1, H), W2, b2.reshape(1, O))


# speedup vs baseline: 4.1006x; 1.0417x over previous
import functools

import jax
import jax.numpy as jnp
from jax import lax
from jax.experimental import pallas as pl
from jax.experimental.pallas import tpu as pltpu
from jax.experimental.pallas import tpu_sc as plsc

B = 16384
H = 128
O = 64
T1_ROWS = 3 * 6 * 6
T2_ROWS = 31 * 26

NC, NS = 2, 16
NW = NC * NS
BPW = B // NW
CHUNK = 128
NCH = BPW // CHUNK


def _tables_body(eg, ei, em, er, el, w1, b1, t1, t2):
    f32 = jnp.float32
    mg = jnp.dot(eg[...], w1[0:8, :], preferred_element_type=f32)
    mi = jnp.dot(ei[...], w1[8:16, :], preferred_element_type=f32)
    mm = jnp.dot(em[...], w1[16:24, :], preferred_element_type=f32)
    mr = jnp.dot(er[...], w1[24:32, :], preferred_element_type=f32)
    ml = jnp.dot(el[...], w1[32:40, :], preferred_element_type=f32)

    r1 = lax.broadcasted_iota(jnp.int32, (T1_ROWS, 1), 0)
    og = (r1 // 36 == lax.broadcasted_iota(jnp.int32, (T1_ROWS, 3), 1)).astype(f32)
    oi = ((r1 // 6) % 6 == lax.broadcasted_iota(jnp.int32, (T1_ROWS, 6), 1)).astype(f32)
    om = (r1 % 6 == lax.broadcasted_iota(jnp.int32, (T1_ROWS, 6), 1)).astype(f32)
    t1[...] = (jnp.dot(og, mg, preferred_element_type=f32)
               + jnp.dot(oi, mi, preferred_element_type=f32)
               + jnp.dot(om, mm, preferred_element_type=f32)
               + b1[...])

    r2 = lax.broadcasted_iota(jnp.int32, (T2_ROWS, 1), 0)
    orr = (r2 // 26 == lax.broadcasted_iota(jnp.int32, (T2_ROWS, 31), 1)).astype(f32)
    ol = (r2 % 26 == lax.broadcasted_iota(jnp.int32, (T2_ROWS, 26), 1)).astype(f32)
    t2[...] = (jnp.dot(orr, mr, preferred_element_type=f32)
               + jnp.dot(ol, ml, preferred_element_type=f32))


_build_tables = pl.pallas_call(
    _tables_body,
    out_shape=(jax.ShapeDtypeStruct((T1_ROWS, H), jnp.float32),
               jax.ShapeDtypeStruct((T2_ROWS, H), jnp.float32)),
)


def _gather_body(g, i, m, r, l, t1, t2, out,
                 gv, iv, mv, rv, lv, idx1, idx2, b1s, b2s, wbs,
                 sidx, sg1, sg2, sw):
    wid = lax.axis_index("s") * NC + lax.axis_index("c")
    base = wid * BPW

    cps = [pltpu.async_copy(src.at[pl.ds(base, BPW)], dst, sidx)
           for src, dst in ((g, gv), (i, iv), (m, mv), (r, rv), (l, lv))]
    for cp in cps:
        cp.wait()

    def icomp(k, carry):
        s = pl.ds(k * 16, 16)
        idx1[s] = gv[s] * 36 + iv[s] * 6 + mv[s]
        idx2[s] = rv[s] * 26 + lv[s]
        return carry
    lax.fori_loop(0, BPW // 16, icomp, 0)

    def fire(c):
        s = c % 2
        sl = pl.ds(c * CHUNK, CHUNK)
        g1 = pltpu.async_copy(t1.at[idx1.at[sl]], b1s.at[s], sg1.at[s])
        g2 = pltpu.async_copy(t2.at[idx2.at[sl]], b2s.at[s], sg2.at[s])
        return g1, g2

    pend = [fire(0), fire(1)]
    wr = [None, None]
    for c in range(NCH):
        s = c % 2
        g1, g2 = pend[s]
        g1.wait()
        g2.wait()
        if wr[s] is not None:
            wr[s].wait()

        def row(rr, carry):
            for j in range(H // 16):
                js = pl.ds(j * 16, 16)
                wbs[s, rr, js] = b1s[s, rr, js] + b2s[s, rr, js]
            return carry
        lax.fori_loop(0, CHUNK, row, 0)

        if c + 2 < NCH:
            pend[s] = fire(c + 2)
        wr[s] = pltpu.async_copy(wbs.at[s], out.at[pl.ds(base + c * CHUNK, CHUNK)],
                                 sw.at[s])
    wr[0].wait()
    wr[1].wait()


@functools.cache
def _make_gather_sum():
  return functools.partial(
    pl.kernel,
    out_type=jax.ShapeDtypeStruct((B, H), jnp.float32),
    mesh=plsc.VectorSubcoreMesh(core_axis_name="c", subcore_axis_name="s",
                                num_cores=NC, num_subcores=NS),
    scratch_types=[
        pltpu.VMEM((BPW,), jnp.int32),
        pltpu.VMEM((BPW,), jnp.int32),
        pltpu.VMEM((BPW,), jnp.int32),
        pltpu.VMEM((BPW,), jnp.int32),
        pltpu.VMEM((BPW,), jnp.int32),
        pltpu.VMEM((BPW,), jnp.int32),
        pltpu.VMEM((BPW,), jnp.int32),
        pltpu.VMEM((2, CHUNK, H), jnp.float32),
        pltpu.VMEM((2, CHUNK, H), jnp.float32),
        pltpu.VMEM((2, CHUNK, H), jnp.float32),
        pltpu.SemaphoreType.DMA,
        pltpu.SemaphoreType.DMA((2,)),
        pltpu.SemaphoreType.DMA((2,)),
        pltpu.SemaphoreType.DMA((2,)),
    ],
  )(_gather_body)


def _mlp_body(hp, sc, w1r, lg, lb, w2, b2, out):
    x = hp[...] + sc[...] * w1r[...]
    mu = jnp.mean(x, axis=-1, keepdims=True)
    d = x - mu
    var = jnp.mean(d * d, axis=-1, keepdims=True)
    y = d * lax.rsqrt(var + 1e-5) * lg[...] + lb[...]
    y = jnp.maximum(y, 0.0)
    out[...] = jnp.dot(y, w2[...], preferred_element_type=jnp.float32) + b2[...]


_MLP_BS = 2048


def _mlp(hp, scal, w1row, ln_g, ln_b, W2, b2):
    grid = (B // _MLP_BS,)
    return pl.pallas_call(
        _mlp_body,
        grid=grid,
        in_specs=[
            pl.BlockSpec((_MLP_BS, H), lambda n: (n, 0)),
            pl.BlockSpec((_MLP_BS, 1), lambda n: (n, 0)),
            pl.BlockSpec((1, H), lambda n: (0, 0)),
            pl.BlockSpec((1, H), lambda n: (0, 0)),
            pl.BlockSpec((1, H), lambda n: (0, 0)),
            pl.BlockSpec((H, O), lambda n: (0, 0)),
            pl.BlockSpec((1, O), lambda n: (0, 0)),
        ],
        out_specs=pl.BlockSpec((_MLP_BS, O), lambda n: (n, 0)),
        out_shape=jax.ShapeDtypeStruct((B, O), jnp.float32),
    )(hp, scal, w1row, ln_g, ln_b, W2, b2)


def kernel(gender, insurance, marital_status, race, language, scalar_inputs,
           emb_gender, emb_insurance, emb_marital_status, emb_race, emb_language,
           W1, b1, ln_g, ln_b, W2, b2):
    g = gender.astype(jnp.int32)
    i = insurance.astype(jnp.int32)
    m = marital_status.astype(jnp.int32)
    r = race.astype(jnp.int32)
    l = language.astype(jnp.int32)

    t1, t2 = _build_tables(emb_gender, emb_insurance, emb_marital_status,
                           emb_race, emb_language, W1, b1.reshape(1, H))
    hp = _make_gather_sum()(g, i, m, r, l, t1, t2)
    return _mlp(hp, scalar_inputs, W1[40].reshape(1, H),
                ln_g.reshape(1, H), ln_b.reshape(1, H), W2, b2.reshape(1, O))
